# Initial kernel scaffold; baseline (speedup 1.0000x reference)
import jax
import jax.numpy as jnp
from jax.experimental import pallas as pl


def kernel(x, mask):
    raise NotImplementedError("write your pallas kernel here")


# speedup vs baseline: 1.1236x; 1.1236x over previous
import jax
import jax.numpy as jnp
from jax.experimental import pallas as pl

_N = 16384
_C = 26
_K = 101
_BLOCK = 512


def _onehot_kernel(x_ref, m_ref, o_ref):
    idx = ((x_ref[...] + 1.0) * m_ref[...]).astype(jnp.int32)
    iota = jax.lax.broadcasted_iota(jnp.int32, (1, _K), 1)
    for c in range(_C):
        col = idx[:, c : c + 1]
        o_ref[:, c * _K : (c + 1) * _K] = (col == iota).astype(jnp.float32)


def kernel(x, mask):
    n, c = x.shape
    grid = (n // _BLOCK,)
    return pl.pallas_call(
        _onehot_kernel,
        grid=grid,
        in_specs=[
            pl.BlockSpec((_BLOCK, c), lambda i: (i, 0)),
            pl.BlockSpec((_BLOCK, c), lambda i: (i, 0)),
        ],
        out_specs=pl.BlockSpec((_BLOCK, c * _K), lambda i: (i, 0)),
        out_shape=jax.ShapeDtypeStruct((n, c * _K), jnp.float32),
    )(x, mask)


# speedup vs baseline: 1.3662x; 1.2160x over previous
import functools

import jax
import jax.numpy as jnp
import numpy as np
from jax.experimental import pallas as pl

_C = 26
_K = 101
_W = _C * _K
_BLOCK = 512


def _onehot_kernel(x_ref, m_ref, s_ref, k_ref, o_ref):
    idx = (x_ref[...] + 1.0) * m_ref[...]
    spread = jnp.dot(idx, s_ref[...], preferred_element_type=jnp.float32)
    o_ref[...] = (spread == k_ref[...]).astype(jnp.float32)


@functools.lru_cache(maxsize=1)
def _constants():
    j = np.arange(_W)
    sel = (j // _K == np.arange(_C)[:, None]).astype(np.float32)
    kvec = (j % _K).astype(np.float32).reshape(1, _W)
    return jnp.asarray(sel), jnp.asarray(kvec)


def kernel(x, mask):
    n, c = x.shape
    sel, kvec = _constants()
    grid = (n // _BLOCK,)
    return pl.pallas_call(
        _onehot_kernel,
        grid=grid,
        in_specs=[
            pl.BlockSpec((_BLOCK, c), lambda i: (i, 0)),
            pl.BlockSpec((_BLOCK, c), lambda i: (i, 0)),
            pl.BlockSpec((_C, _W), lambda i: (0, 0)),
            pl.BlockSpec((1, _W), lambda i: (0, 0)),
        ],
        out_specs=pl.BlockSpec((_BLOCK, _W), lambda i: (i, 0)),
        out_shape=jax.ShapeDtypeStruct((n, _W), jnp.float32),
    )(x, mask, sel, kvec)


# speedup vs baseline: 1.4117x; 1.0333x over previous
import functools

import jax
import jax.numpy as jnp
import numpy as np
from jax.experimental import pallas as pl

_C = 26
_K = 101
_W = _C * _K
_BLOCK = 1024


def _onehot_kernel(x_ref, m_ref, s_ref, k_ref, o_ref):
    idx = (x_ref[...] + 1.0) * m_ref[...]
    spread = jnp.dot(idx, s_ref[...], preferred_element_type=jnp.float32)
    o_ref[...] = (spread == k_ref[...]).astype(jnp.float32)


@functools.lru_cache(maxsize=1)
def _constants():
    j = np.arange(_W)
    sel = (j // _K == np.arange(_C)[:, None]).astype(np.float32)
    kvec = (j % _K).astype(np.float32).reshape(1, _W)
    return jnp.asarray(sel), jnp.asarray(kvec)


def kernel(x, mask):
    n, c = x.shape
    sel, kvec = _constants()
    grid = (n // _BLOCK,)
    return pl.pallas_call(
        _onehot_kernel,
        grid=grid,
        in_specs=[
            pl.BlockSpec((_BLOCK, c), lambda i: (i, 0)),
            pl.BlockSpec((_BLOCK, c), lambda i: (i, 0)),
            pl.BlockSpec((_C, _W), lambda i: (0, 0)),
            pl.BlockSpec((1, _W), lambda i: (0, 0)),
        ],
        out_specs=pl.BlockSpec((_BLOCK, _W), lambda i: (i, 0)),
        out_shape=jax.ShapeDtypeStruct((n, _W), jnp.float32),
    )(x, mask, sel, kvec)
